# baseline (speedup 1.0000x reference)
import functools

import jax
import jax.numpy as jnp
from jax.experimental import pallas as pl
from jax.experimental.pallas import tpu as pltpu

_DN_LT = (((0,), (0,)), ((), ()))
_DN_RT = (((1,), (1,)), ((), ()))
_LOG2E = 1.4426950408889634


def _gat_kernel(x_ref, adj_ref, W_ref, asrc_ref, adst_ref, bias_ref, out_ref,
                h_s, esc_s, edr_s, *, num_heads, f_per_head, tile_n):
    t = pl.program_id(1)
    hf = num_heads * f_per_head

    @pl.when(t == 0)
    def _prep():
        x_b = x_ref[0]
        h = jnp.dot(x_b, W_ref[:], preferred_element_type=jnp.float32)
        h_s[:] = h
        asrc_row = jnp.concatenate(
            [asrc_ref[g:g + 1, :] for g in range(num_heads)], axis=1)
        adst_row = jnp.concatenate(
            [adst_ref[g:g + 1, :] for g in range(num_heads)], axis=1)
        rowg = jax.lax.broadcasted_iota(jnp.int32, (num_heads, hf), 0)
        colg = (jax.lax.broadcasted_iota(jnp.int32, (num_heads, hf), 1)
                // f_per_head)
        blk = rowg == colg
        a_src_bdT = jnp.where(blk, asrc_row, 0.0)
        a_dst_bdT = jnp.where(blk, adst_row, 0.0)
        log2e = jnp.float32(_LOG2E)
        esc_s[:] = jax.lax.dot_general(
            h, a_src_bdT, _DN_RT, preferred_element_type=jnp.float32) * log2e
        edr_s[:] = jax.lax.dot_general(
            a_dst_bdT, h, _DN_RT, preferred_element_type=jnp.float32) * log2e

    mask = adj_ref[0] > 0.5
    esc = esc_s[:]
    j0 = t * tile_n
    outs, den_rows = [], []
    for hd in range(num_heads):
        edr_row = edr_s[hd:hd + 1, pl.ds(j0, tile_n)]
        q = esc[:, hd:hd + 1] + edr_row
        q = jnp.maximum(q, 0.2 * q)
        m = jnp.max(q, axis=0, keepdims=True)
        ex = jnp.exp2(q - m)
        exm = jnp.where(mask, ex, 0.0)
        den_rows.append(jnp.sum(exm, axis=0, keepdims=True))
        outs.append(jax.lax.dot_general(
            exm, h_s[:, hd * f_per_head:(hd + 1) * f_per_head], _DN_LT,
            preferred_element_type=jnp.float32))

    dens = jnp.concatenate(den_rows, axis=0)
    ident_h = (jax.lax.broadcasted_iota(jnp.int32, (num_heads, num_heads), 0)
               == jax.lax.broadcasted_iota(jnp.int32, (num_heads, num_heads), 1)
               ).astype(jnp.float32)
    densT = jax.lax.dot_general(dens, ident_h, _DN_LT,
                                preferred_element_type=jnp.float32)
    recip = 1.0 / (densT + 1e-16)
    head_ones = (jax.lax.broadcasted_iota(jnp.int32, (num_heads, hf), 0)
                 == (jax.lax.broadcasted_iota(jnp.int32, (num_heads, hf), 1)
                     // f_per_head)).astype(jnp.float32)
    rep = jnp.dot(recip, head_ones,
                  preferred_element_type=jnp.float32)
    out_ref[0] = (jnp.concatenate(outs, axis=1) * rep
                  + jnp.reshape(bias_ref[:], (1, hf)))


def kernel(x, adj, W, a_src, a_dst, bias):
    B, N, Din = x.shape
    H, F = a_src.shape
    HF = H * F
    TILE_N = 128

    return pl.pallas_call(
        functools.partial(_gat_kernel, num_heads=H, f_per_head=F,
                          tile_n=TILE_N),
        grid=(B, N // TILE_N),
        in_specs=[
            pl.BlockSpec((1, N, Din), lambda b, t: (b, 0, 0)),
            pl.BlockSpec((1, N, TILE_N), lambda b, t: (b, 0, t)),
            pl.BlockSpec((Din, HF), lambda b, t: (0, 0)),
            pl.BlockSpec((H, F), lambda b, t: (0, 0)),
            pl.BlockSpec((H, F), lambda b, t: (0, 0)),
            pl.BlockSpec((HF,), lambda b, t: (0,)),
        ],
        out_specs=pl.BlockSpec((1, TILE_N, HF), lambda b, t: (b, t, 0)),
        out_shape=jax.ShapeDtypeStruct((B, N, HF), x.dtype),
        scratch_shapes=[
            pltpu.VMEM((N, HF), jnp.float32),
            pltpu.VMEM((N, H), jnp.float32),
            pltpu.VMEM((H, N), jnp.float32),
        ],
        compiler_params=pltpu.CompilerParams(
            dimension_semantics=("parallel", "arbitrary")),
    )(x, adj, W, a_src, a_dst, bias)


# speedup vs baseline: 1.4001x; 1.4001x over previous
import functools

import jax
import jax.numpy as jnp
from jax.experimental import pallas as pl
from jax.experimental.pallas import tpu as pltpu

_DN_LT = (((0,), (0,)), ((), ()))
_DN_RT = (((1,), (1,)), ((), ()))
_LOG2E = 1.4426950408889634


def _gat_kernel(x_ref, adj_ref, W_ref, asrc_ref, adst_ref, bias_ref, out_ref,
                h_s, esc_s, edr_s, *, num_heads, f_per_head, tile_n):
    t = pl.program_id(1)
    hf = num_heads * f_per_head

    @pl.when(t == 0)
    def _prep():
        x_b = x_ref[0]
        h = jnp.dot(x_b, W_ref[:], preferred_element_type=jnp.float32)
        h_s[:] = h
        asrc_row = jnp.concatenate(
            [asrc_ref[g:g + 1, :] for g in range(num_heads)], axis=1)
        adst_row = jnp.concatenate(
            [adst_ref[g:g + 1, :] for g in range(num_heads)], axis=1)
        rowg = jax.lax.broadcasted_iota(jnp.int32, (num_heads, hf), 0)
        colg = (jax.lax.broadcasted_iota(jnp.int32, (num_heads, hf), 1)
                // f_per_head)
        blk = rowg == colg
        a_src_bdT = jnp.where(blk, asrc_row, 0.0)
        a_dst_bdT = jnp.where(blk, adst_row, 0.0)
        log2e = jnp.float32(_LOG2E)
        esc_s[:] = jax.lax.dot_general(
            h, a_src_bdT, _DN_RT, preferred_element_type=jnp.float32) * log2e
        edr = jax.lax.dot_general(
            a_dst_bdT, h, _DN_RT, preferred_element_type=jnp.float32) * log2e
        for tt in range(edr_s.shape[0]):
            edr_s[tt] = edr[:, tt * tile_n:(tt + 1) * tile_n]

    mask = adj_ref[0] > 0.5
    esc = esc_s[:]
    edr_t = edr_s[t]
    outs, den_rows = [], []
    for hd in range(num_heads):
        edr_row = edr_t[hd:hd + 1, :]
        q = esc[:, hd:hd + 1] + edr_row
        q = jnp.maximum(q, 0.2 * q)
        m = jnp.max(q, axis=0, keepdims=True)
        ex = jnp.exp2(q - m)
        exm = jnp.where(mask, ex, 0.0)
        den_rows.append(jnp.sum(exm, axis=0, keepdims=True))
        outs.append(jax.lax.dot_general(
            exm, h_s[:, hd * f_per_head:(hd + 1) * f_per_head], _DN_LT,
            preferred_element_type=jnp.float32))

    dens = jnp.concatenate(den_rows, axis=0)
    ident_h = (jax.lax.broadcasted_iota(jnp.int32, (num_heads, num_heads), 0)
               == jax.lax.broadcasted_iota(jnp.int32, (num_heads, num_heads), 1)
               ).astype(jnp.float32)
    densT = jax.lax.dot_general(dens, ident_h, _DN_LT,
                                preferred_element_type=jnp.float32)
    recip = 1.0 / (densT + 1e-16)
    head_ones = (jax.lax.broadcasted_iota(jnp.int32, (num_heads, hf), 0)
                 == (jax.lax.broadcasted_iota(jnp.int32, (num_heads, hf), 1)
                     // f_per_head)).astype(jnp.float32)
    rep = jnp.dot(recip, head_ones,
                  preferred_element_type=jnp.float32)
    out_ref[0] = (jnp.concatenate(outs, axis=1) * rep
                  + jnp.reshape(bias_ref[:], (1, hf)))


def kernel(x, adj, W, a_src, a_dst, bias):
    B, N, Din = x.shape
    H, F = a_src.shape
    HF = H * F
    TILE_N = 256

    return pl.pallas_call(
        functools.partial(_gat_kernel, num_heads=H, f_per_head=F,
                          tile_n=TILE_N),
        grid=(B, N // TILE_N),
        in_specs=[
            pl.BlockSpec((1, N, Din), lambda b, t: (b, 0, 0)),
            pl.BlockSpec((1, N, TILE_N), lambda b, t: (b, 0, t)),
            pl.BlockSpec((Din, HF), lambda b, t: (0, 0)),
            pl.BlockSpec((H, F), lambda b, t: (0, 0)),
            pl.BlockSpec((H, F), lambda b, t: (0, 0)),
            pl.BlockSpec((HF,), lambda b, t: (0,)),
        ],
        out_specs=pl.BlockSpec((1, TILE_N, HF), lambda b, t: (b, t, 0)),
        out_shape=jax.ShapeDtypeStruct((B, N, HF), x.dtype),
        scratch_shapes=[
            pltpu.VMEM((N, HF), jnp.float32),
            pltpu.VMEM((N, H), jnp.float32),
            pltpu.VMEM((N // TILE_N, H, TILE_N), jnp.float32),
        ],
        compiler_params=pltpu.CompilerParams(
            dimension_semantics=("parallel", "arbitrary")),
    )(x, adj, W, a_src, a_dst, bias)


# speedup vs baseline: 1.5717x; 1.1225x over previous
import functools

import jax
import jax.numpy as jnp
from jax.experimental import pallas as pl
from jax.experimental.pallas import tpu as pltpu

_DN_LT = (((0,), (0,)), ((), ()))
_DN_RT = (((1,), (1,)), ((), ()))


def _gat_kernel(x_ref, adj_ref, W_ref, asrc_ref, adst_ref, bias_ref, out_ref,
                *, num_heads, f_per_head):
    x_b = x_ref[0]
    mask = adj_ref[0] > 0.5
    hf = num_heads * f_per_head

    h = jnp.dot(x_b, W_ref[:], preferred_element_type=jnp.float32)

    asrc_row = jnp.concatenate(
        [asrc_ref[g:g + 1, :] for g in range(num_heads)], axis=1)
    adst_row = jnp.concatenate(
        [adst_ref[g:g + 1, :] for g in range(num_heads)], axis=1)
    rowg = jax.lax.broadcasted_iota(jnp.int32, (num_heads, hf), 0)
    colg = jax.lax.broadcasted_iota(jnp.int32, (num_heads, hf), 1) // f_per_head
    blk = rowg == colg
    a_src_bdT = jnp.where(blk, asrc_row, 0.0)
    a_dst_bdT = jnp.where(blk, adst_row, 0.0)

    log2e = jnp.float32(1.4426950408889634)
    esc = jax.lax.dot_general(h, a_src_bdT, _DN_RT,
                              preferred_element_type=jnp.float32) * log2e
    edr = jax.lax.dot_general(a_dst_bdT, h, _DN_RT,
                              preferred_element_type=jnp.float32) * log2e

    outs, den_rows = [], []
    for hd in range(num_heads):
        q = esc[:, hd:hd + 1] + edr[hd:hd + 1, :]
        q = jnp.maximum(q, 0.2 * q)
        m = jnp.max(q, axis=0, keepdims=True)
        ex = jnp.exp2(q - m)
        exm = jnp.where(mask, ex, 0.0)
        den_rows.append(jnp.sum(exm, axis=0, keepdims=True))
        outs.append(jax.lax.dot_general(
            exm, h[:, hd * f_per_head:(hd + 1) * f_per_head], _DN_LT,
            preferred_element_type=jnp.float32))
    dens = jnp.concatenate(den_rows, axis=0)
    ident_h = (jax.lax.broadcasted_iota(jnp.int32, (num_heads, num_heads), 0)
               == jax.lax.broadcasted_iota(jnp.int32, (num_heads, num_heads), 1)
               ).astype(jnp.float32)
    densT = jax.lax.dot_general(dens, ident_h, _DN_LT,
                                preferred_element_type=jnp.float32)
    recip = 1.0 / (densT + 1e-16)
    rep = jnp.dot(recip, blk.astype(jnp.float32),
                  preferred_element_type=jnp.float32)
    out_ref[0] = (jnp.concatenate(outs, axis=1) * rep
                  + jnp.reshape(bias_ref[:], (1, hf)))


def kernel(x, adj, W, a_src, a_dst, bias):
    B, N, Din = x.shape
    H, F = a_src.shape
    HF = H * F

    return pl.pallas_call(
        functools.partial(_gat_kernel, num_heads=H, f_per_head=F),
        grid=(B,),
        in_specs=[
            pl.BlockSpec((1, N, Din), lambda b: (b, 0, 0)),
            pl.BlockSpec((1, N, N), lambda b: (b, 0, 0)),
            pl.BlockSpec((Din, HF), lambda b: (0, 0)),
            pl.BlockSpec((H, F), lambda b: (0, 0)),
            pl.BlockSpec((H, F), lambda b: (0, 0)),
            pl.BlockSpec((HF,), lambda b: (0,)),
        ],
        out_specs=pl.BlockSpec((1, N, HF), lambda b: (b, 0, 0)),
        out_shape=jax.ShapeDtypeStruct((B, N, HF), x.dtype),
        compiler_params=pltpu.CompilerParams(
            dimension_semantics=("parallel",)),
    )(x, adj, W, a_src, a_dst, bias)
